# Initial kernel scaffold; baseline (speedup 1.0000x reference)
import jax
import jax.numpy as jnp
from jax.experimental import pallas as pl


def kernel(x, Wc, temperature, genes, Wd, Wu, rms_w, scale):
    raise NotImplementedError("write your pallas kernel here")


# speedup vs baseline: 4.5335x; 4.5335x over previous
import jax
import jax.numpy as jnp
from jax.experimental import pallas as pl
from jax.experimental.pallas import tpu as pltpu

EPS = 1e-6
N_ACTIVE = 8
TILE = 512


def _fold_kernel(genes_ref, wd_ref, wu_ref, gdu_ref):
    g = genes_ref[...]
    d = wd_ref.shape[0]
    gdu_ref[:, :d] = jax.lax.dot_general(
        g, wd_ref[...], (((1,), (1,)), ((), ())),
        preferred_element_type=jnp.float32)
    gdu_ref[:, d:] = jax.lax.dot_general(
        g, wu_ref[...], (((1,), (1,)), ((), ())),
        preferred_element_type=jnp.float32)


def _dna_kernel(params_ref, x_ref, wc_ref, gdu_ref, rmsw_ref, out_ref):
    tinv = params_ref[0]
    scale = params_ref[1]
    d = x_ref.shape[-1]
    x = x_ref[...]
    xn = x * jax.lax.rsqrt(jnp.mean(x * x, axis=1, keepdims=True) + EPS)
    xn = xn * rmsw_ref[...]
    logits = jax.lax.dot_general(
        xn, wc_ref[...], (((1,), (1,)), ((), ())),
        preferred_element_type=jnp.float32) * tinv

    n_genes = logits.shape[-1]
    col = jax.lax.broadcasted_iota(jnp.int32, logits.shape, 1)
    work = logits
    sel = jnp.zeros(logits.shape, dtype=jnp.bool_)
    for _ in range(N_ACTIVE):
        m = jnp.max(work, axis=1, keepdims=True)
        elig = work == m
        first = jnp.min(jnp.where(elig, col, n_genes), axis=1, keepdims=True)
        pick = col == first
        sel = jnp.logical_or(sel, pick)
        work = jnp.where(pick, -jnp.inf, work)
    zm = jnp.max(jnp.where(sel, logits, -jnp.inf), axis=1, keepdims=True)
    e = jnp.where(sel, jnp.exp(logits - zm), 0.0)
    expr = e / jnp.sum(e, axis=1, keepdims=True)

    h = jnp.dot(expr, gdu_ref[...], preferred_element_type=jnp.float32)
    gate = jnp.tanh(h[:, :d]) * xn
    out_ref[...] = gate * h[:, d:] * scale


def kernel(x, Wc, temperature, genes, Wd, Wu, rms_w, scale):
    b, t, d = x.shape
    n_genes, gene_dim = genes.shape
    n_tok = b * t
    xf = x.reshape(n_tok, d)

    gdu = pl.pallas_call(
        _fold_kernel,
        out_shape=jax.ShapeDtypeStruct((n_genes, 2 * d), jnp.float32),
    )(genes, Wd, Wu)

    tinv = 1.0 / jnp.maximum(temperature, 0.1)
    params = jnp.stack([tinv, scale]).astype(jnp.float32)

    out = pl.pallas_call(
        _dna_kernel,
        grid=(n_tok // TILE,),
        in_specs=[
            pl.BlockSpec(memory_space=pltpu.SMEM),
            pl.BlockSpec((TILE, d), lambda i: (i, 0)),
            pl.BlockSpec((n_genes, d), lambda i: (0, 0)),
            pl.BlockSpec((n_genes, 2 * d), lambda i: (0, 0)),
            pl.BlockSpec((1, d), lambda i: (0, 0)),
        ],
        out_specs=pl.BlockSpec((TILE, d), lambda i: (i, 0)),
        out_shape=jax.ShapeDtypeStruct((n_tok, d), jnp.float32),
    )(params, xf, Wc, gdu, rms_w.reshape(1, d))
    return out.reshape(b, t, d)


# speedup vs baseline: 6.9388x; 1.5306x over previous
import jax
import jax.numpy as jnp
from jax.experimental import pallas as pl
from jax.experimental.pallas import tpu as pltpu

EPS = 1e-6
N_ACTIVE = 8
TILE = 512


def _fold_kernel(params_ref, genes_ref, wd_ref, wu_ref, wc_ref, rmsw_ref,
                 wc2_ref, gdu_ref):
    tinv = params_ref[0]
    scale = params_ref[1]
    d = wd_ref.shape[0]
    g = genes_ref[...]
    rw = rmsw_ref[...]
    wc2_ref[...] = wc_ref[...] * (rw * tinv)
    gdu_ref[:, :d] = jax.lax.dot_general(
        g, wd_ref[...], (((1,), (1,)), ((), ())),
        preferred_element_type=jnp.float32)
    gdu_ref[:, d:] = jax.lax.dot_general(
        g, wu_ref[...], (((1,), (1,)), ((), ())),
        preferred_element_type=jnp.float32) * (rw * scale)


def _dna_kernel(x_ref, wc2_ref, gdu_ref, out_ref):
    d = x_ref.shape[-1]
    x = x_ref[...]
    xr = x * jax.lax.rsqrt(jnp.mean(x * x, axis=1, keepdims=True) + EPS)
    lg = jax.lax.dot_general(
        wc2_ref[...], xr, (((1,), (1,)), ((), ())),
        preferred_element_type=jnp.float32)

    work = lg
    sel = jnp.zeros(lg.shape, dtype=jnp.bool_)
    zm = None
    for i in range(N_ACTIVE):
        m = jnp.max(work, axis=0, keepdims=True)
        if i == 0:
            zm = m
        pick = work == m
        sel = jnp.logical_or(sel, pick)
        work = jnp.where(pick, -jnp.inf, work)
    e = jnp.where(sel, jnp.exp(lg - zm), 0.0)
    p = e / jnp.sum(e, axis=0, keepdims=True)

    h = jax.lax.dot_general(
        p, gdu_ref[...], (((0,), (0,)), ((), ())),
        preferred_element_type=jnp.float32)
    out_ref[...] = jnp.tanh(h[:, :d]) * xr * h[:, d:]


def kernel(x, Wc, temperature, genes, Wd, Wu, rms_w, scale):
    b, t, d = x.shape
    n_genes, gene_dim = genes.shape
    n_tok = b * t
    xf = x.reshape(n_tok, d)

    tinv = 1.0 / jnp.maximum(temperature, 0.1)
    params = jnp.stack([tinv, scale]).astype(jnp.float32)

    wc2, gdu = pl.pallas_call(
        _fold_kernel,
        in_specs=[
            pl.BlockSpec(memory_space=pltpu.SMEM),
            pl.BlockSpec(memory_space=pltpu.VMEM),
            pl.BlockSpec(memory_space=pltpu.VMEM),
            pl.BlockSpec(memory_space=pltpu.VMEM),
            pl.BlockSpec(memory_space=pltpu.VMEM),
            pl.BlockSpec(memory_space=pltpu.VMEM),
        ],
        out_shape=(
            jax.ShapeDtypeStruct((n_genes, d), jnp.float32),
            jax.ShapeDtypeStruct((n_genes, 2 * d), jnp.float32),
        ),
    )(params, genes, Wd, Wu, Wc, rms_w.reshape(1, d))

    out = pl.pallas_call(
        _dna_kernel,
        grid=(n_tok // TILE,),
        in_specs=[
            pl.BlockSpec((TILE, d), lambda i: (i, 0)),
            pl.BlockSpec((n_genes, d), lambda i: (0, 0)),
            pl.BlockSpec((n_genes, 2 * d), lambda i: (0, 0)),
        ],
        out_specs=pl.BlockSpec((TILE, d), lambda i: (i, 0)),
        out_shape=jax.ShapeDtypeStruct((n_tok, d), jnp.float32),
    )(xf, wc2, gdu)
    return out.reshape(b, t, d)


# speedup vs baseline: 7.4517x; 1.0739x over previous
import jax
import jax.numpy as jnp
from jax.experimental import pallas as pl
from jax.experimental.pallas import tpu as pltpu

EPS = 1e-6
N_ACTIVE = 8
TILE = 1024


def _fold_kernel(params_ref, genes_ref, wd_ref, wu_ref, wc_ref, rmsw_ref,
                 wc2_ref, gdu_ref):
    tinv = params_ref[0]
    scale = params_ref[1]
    d = wd_ref.shape[0]
    g = genes_ref[...]
    rw = rmsw_ref[...]
    wc2_ref[...] = wc_ref[...] * (rw * tinv)
    gdu_ref[:, :d] = jax.lax.dot_general(
        g, wd_ref[...], (((1,), (1,)), ((), ())),
        preferred_element_type=jnp.float32)
    gdu_ref[:, d:] = jax.lax.dot_general(
        g, wu_ref[...], (((1,), (1,)), ((), ())),
        preferred_element_type=jnp.float32) * (rw * scale)


def _dna_kernel(x_ref, wc2_ref, gdu_ref, out_ref):
    x = x_ref[...]
    wc2 = wc2_ref[...]
    gdu = gdu_ref[...]
    d = x.shape[-1]
    rs = jax.lax.rsqrt(jnp.mean(x * x, axis=1, keepdims=True) + EPS)
    xr = x * rs
    lg = jax.lax.dot_general(
        wc2, xr, (((1,), (1,)), ((), ())),
        preferred_element_type=jnp.float32)

    work = lg
    sel = jnp.zeros(lg.shape, dtype=jnp.bool_)
    zm = None
    for i in range(N_ACTIVE):
        m = jnp.max(work, axis=0, keepdims=True)
        if i == 0:
            zm = m
        pick = work == m
        sel = jnp.logical_or(sel, pick)
        work = jnp.where(pick, -jnp.inf, work)
    e = jnp.where(sel, jnp.exp(lg - zm), 0.0)
    p = e / jnp.sum(e, axis=0, keepdims=True)

    h = jax.lax.dot_general(
        p, gdu, (((0,), (0,)), ((), ())),
        preferred_element_type=jnp.float32)
    out_ref[...] = jnp.tanh(h[:, :d]) * xr * h[:, d:]


def kernel(x, Wc, temperature, genes, Wd, Wu, rms_w, scale):
    b, t, d = x.shape
    n_genes, gene_dim = genes.shape
    n_tok = b * t
    xf = x.reshape(n_tok, d)

    tinv = 1.0 / jnp.maximum(temperature, 0.1)
    params = jnp.stack([tinv, scale]).astype(jnp.float32)

    wc2, gdu = pl.pallas_call(
        _fold_kernel,
        in_specs=[
            pl.BlockSpec(memory_space=pltpu.SMEM),
            pl.BlockSpec(memory_space=pltpu.VMEM),
            pl.BlockSpec(memory_space=pltpu.VMEM),
            pl.BlockSpec(memory_space=pltpu.VMEM),
            pl.BlockSpec(memory_space=pltpu.VMEM),
            pl.BlockSpec(memory_space=pltpu.VMEM),
        ],
        out_shape=(
            jax.ShapeDtypeStruct((n_genes, d), jnp.float32),
            jax.ShapeDtypeStruct((n_genes, 2 * d), jnp.float32),
        ),
    )(params, genes, Wd, Wu, Wc, rms_w.reshape(1, d))

    out = pl.pallas_call(
        _dna_kernel,
        grid=(n_tok // TILE,),
        in_specs=[
            pl.BlockSpec((TILE, d), lambda i: (i, 0)),
            pl.BlockSpec((n_genes, d), lambda i: (0, 0)),
            pl.BlockSpec((n_genes, 2 * d), lambda i: (0, 0)),
        ],
        out_specs=pl.BlockSpec((TILE, d), lambda i: (i, 0)),
        out_shape=jax.ShapeDtypeStruct((n_tok, d), jnp.float32),
    )(xf, wc2, gdu)
    return out.reshape(b, t, d)


# speedup vs baseline: 7.4758x; 1.0032x over previous
import jax
import jax.numpy as jnp
from jax.experimental import pallas as pl
from jax.experimental.pallas import tpu as pltpu

EPS = 1e-6
N_ACTIVE = 8
TILE = 1024


def _fold_kernel(params_ref, genes_ref, wd_ref, wu_ref, wc_ref, rmsw_ref,
                 wc2_ref, gdu_ref):
    tinv = params_ref[0]
    scale = params_ref[1]
    d = wd_ref.shape[0]
    g = genes_ref[...]
    rw = rmsw_ref[...]
    wc2_ref[...] = wc_ref[...] * (rw * tinv)
    gdu_ref[:, :d] = jax.lax.dot_general(
        g, wd_ref[...], (((1,), (1,)), ((), ())),
        preferred_element_type=jnp.float32)
    gdu_ref[:, d:] = jax.lax.dot_general(
        g, wu_ref[...], (((1,), (1,)), ((), ())),
        preferred_element_type=jnp.float32) * (rw * scale)


def _dna_kernel(x_ref, wc2_ref, gdu_ref, out_ref):
    x = x_ref[...]
    wc2 = wc2_ref[...]
    gdu = gdu_ref[...]
    d = x.shape[-1]
    rs = jax.lax.rsqrt(jnp.mean(x * x, axis=1, keepdims=True) + EPS)
    xr = x * rs
    lg = jax.lax.dot_general(
        wc2, xr, (((1,), (1,)), ((), ())),
        preferred_element_type=jnp.float32)

    work = lg
    sel = jnp.zeros(lg.shape, dtype=jnp.bool_)
    zm = None
    for i in range(N_ACTIVE):
        m = jnp.max(work, axis=0, keepdims=True)
        if i == 0:
            zm = m
        pick = work == m
        sel = jnp.logical_or(sel, pick)
        work = jnp.where(pick, -jnp.inf, work)
    e = jnp.where(sel, jnp.exp(lg - zm), 0.0)
    p = e / jnp.sum(e, axis=0, keepdims=True)

    h = jax.lax.dot_general(
        p, gdu, (((0,), (0,)), ((), ())),
        preferred_element_type=jnp.float32)
    out_ref[...] = jnp.tanh(h[:, :d]) * xr * h[:, d:]


def kernel(x, Wc, temperature, genes, Wd, Wu, rms_w, scale):
    b, t, d = x.shape
    n_genes, gene_dim = genes.shape
    n_tok = b * t
    xf = x.reshape(n_tok, d)

    tinv = 1.0 / jnp.maximum(temperature, 0.1)
    params = jnp.stack([tinv, scale]).astype(jnp.float32)

    wc2, gdu = pl.pallas_call(
        _fold_kernel,
        in_specs=[
            pl.BlockSpec(memory_space=pltpu.SMEM),
            pl.BlockSpec(memory_space=pltpu.VMEM),
            pl.BlockSpec(memory_space=pltpu.VMEM),
            pl.BlockSpec(memory_space=pltpu.VMEM),
            pl.BlockSpec(memory_space=pltpu.VMEM),
            pl.BlockSpec(memory_space=pltpu.VMEM),
        ],
        out_shape=(
            jax.ShapeDtypeStruct((n_genes, d), jnp.float32),
            jax.ShapeDtypeStruct((n_genes, 2 * d), jnp.float32),
        ),
    )(params, genes, Wd, Wu, Wc, rms_w.reshape(1, d))

    out = pl.pallas_call(
        _dna_kernel,
        grid=(n_tok // TILE,),
        in_specs=[
            pl.BlockSpec((TILE, d), lambda i: (i, 0)),
            pl.BlockSpec((n_genes, d), lambda i: (0, 0)),
            pl.BlockSpec((n_genes, 2 * d), lambda i: (0, 0)),
        ],
        out_specs=pl.BlockSpec((TILE, d), lambda i: (i, 0)),
        out_shape=jax.ShapeDtypeStruct((n_tok, d), jnp.float32),
        compiler_params=pltpu.CompilerParams(
            dimension_semantics=("parallel",)),
    )(xf, wc2, gdu)
    return out.reshape(b, t, d)
